# Initial kernel scaffold; baseline (speedup 1.0000x reference)
import jax
import jax.numpy as jnp
from jax.experimental import pallas as pl


def kernel(x, w, is_training):
    raise NotImplementedError("write your pallas kernel here")


# speedup vs baseline: 3.8645x; 3.8645x over previous
import jax
import jax.numpy as jnp
from jax.experimental import pallas as pl
from jax.experimental.pallas import tpu as pltpu

_R = 2048
_E = 256
_EPS = 1e-10
_COMMIT = 0.25


def _vq_tile(x_ref, w_ref, wt_ref,
             qst_ref, enc_ref, idx_ref, loss_ref, perp_ref,
             hist_ref, sse_ref, w2_ref, wbt_ref, lane_ref):
    t = pl.program_id(0)
    n_rows = pl.num_programs(0) * _R

    @pl.when(t == 0)
    def _init():
        w = w_ref[...]
        w2_ref[...] = jnp.sum(w * w, axis=0, keepdims=True)
        wbt_ref[...] = wt_ref[...].astype(jnp.bfloat16)
        hist_ref[...] = jnp.zeros_like(hist_ref)
        sse_ref[0, 0] = 0.0
        lane_ref[...] = jax.lax.broadcasted_iota(
            jnp.int32, (_R, _E), 1).astype(jnp.float32)

    x = x_ref[...]
    xw = jnp.dot(x.astype(jnp.bfloat16), w_ref[...].astype(jnp.bfloat16),
                 preferred_element_type=jnp.float32)
    d = w2_ref[...] - 2.0 * xw

    dmin = jnp.min(d, axis=1, keepdims=True)
    lane = lane_ref[...]
    idx = jnp.min(jnp.where(d == dmin, lane, _E), axis=1, keepdims=True)
    enc = (lane == idx).astype(jnp.float32)

    enc_b = enc.astype(jnp.bfloat16)
    q = jnp.dot(enc_b, wbt_ref[...],
                preferred_element_type=jnp.float32)
    qst_ref[...] = q
    enc_ref[...] = enc
    idx_ref[...] = idx.astype(jnp.int32)

    ones_b = jnp.ones((1, _R), jnp.bfloat16)
    hist_ref[...] += jnp.dot(ones_b, enc_b, preferred_element_type=jnp.float32)
    sse_ref[0, 0] += jnp.sum(x * x) + jnp.sum(dmin)

    @pl.when(t == pl.num_programs(0) - 1)
    def _finish():
        mse = sse_ref[0, 0] / (n_rows * _E)
        loss_ref[...] = jnp.broadcast_to(mse + _COMMIT * mse, (1, 1))
        p = hist_ref[...] / n_rows
        ent = -jnp.sum(p * jnp.log(p + _EPS), keepdims=True)
        perp_ref[...] = jnp.exp(ent).reshape(1, 1)


def kernel(x, w, is_training):
    lead_shape = x.shape[:-1]
    xf = x.reshape(-1, _E)
    n = xf.shape[0]
    grid = n // _R

    qst, enc, idx, loss, perp = pl.pallas_call(
        _vq_tile,
        grid=(grid,),
        in_specs=[
            pl.BlockSpec((_R, _E), lambda t: (t, 0)),
            pl.BlockSpec((_E, _E), lambda t: (0, 0)),
            pl.BlockSpec((_E, _E), lambda t: (0, 0)),
        ],
        out_specs=[
            pl.BlockSpec((_R, _E), lambda t: (t, 0)),
            pl.BlockSpec((_R, _E), lambda t: (t, 0)),
            pl.BlockSpec((_R, 1), lambda t: (t, 0)),
            pl.BlockSpec((1, 1), lambda t: (0, 0)),
            pl.BlockSpec((1, 1), lambda t: (0, 0)),
        ],
        out_shape=[
            jax.ShapeDtypeStruct((n, _E), jnp.float32),
            jax.ShapeDtypeStruct((n, _E), jnp.float32),
            jax.ShapeDtypeStruct((n, 1), jnp.int32),
            jax.ShapeDtypeStruct((1, 1), jnp.float32),
            jax.ShapeDtypeStruct((1, 1), jnp.float32),
        ],
        scratch_shapes=[
            pltpu.VMEM((1, _E), jnp.float32),
            pltpu.SMEM((1, 1), jnp.float32),
            pltpu.VMEM((1, _E), jnp.float32),
            pltpu.VMEM((_E, _E), jnp.bfloat16),
            pltpu.VMEM((_R, _E), jnp.float32),
        ],
    )(xf, w, w.T)

    return (qst.reshape(x.shape), loss[0, 0], perp[0, 0], enc,
            idx.reshape(lead_shape))


# speedup vs baseline: 4.1263x; 1.0678x over previous
import jax
import jax.numpy as jnp
from jax.experimental import pallas as pl
from jax.experimental.pallas import tpu as pltpu

_R = 4096
_E = 256
_EPS = 1e-10
_COMMIT = 0.25


def _vq_tile(x_ref, w_ref, wt_ref,
             qst_ref, enc_ref, idx_ref, loss_ref, perp_ref,
             hist_ref, sse_ref, w2_ref, wbt_ref, lane_ref):
    t = pl.program_id(0)
    n_rows = pl.num_programs(0) * _R

    @pl.when(t == 0)
    def _init():
        w = w_ref[...]
        w2_ref[...] = jnp.sum(w * w, axis=0, keepdims=True)
        wbt_ref[...] = wt_ref[...].astype(jnp.bfloat16)
        hist_ref[...] = jnp.zeros_like(hist_ref)
        sse_ref[0, 0] = 0.0
        lane_ref[...] = jax.lax.broadcasted_iota(
            jnp.int32, (_R, _E), 1).astype(jnp.float32)

    x = x_ref[...]
    xw = jnp.dot(x.astype(jnp.bfloat16), w_ref[...].astype(jnp.bfloat16),
                 preferred_element_type=jnp.float32)
    d = w2_ref[...] - 2.0 * xw

    dmin = jnp.min(d, axis=1, keepdims=True)
    lane = lane_ref[...]
    idx = jnp.min(jnp.where(d == dmin, lane, _E), axis=1, keepdims=True)
    enc = (lane == idx).astype(jnp.float32)

    enc_b = enc.astype(jnp.bfloat16)
    q = jnp.dot(enc_b, wbt_ref[...],
                preferred_element_type=jnp.float32)
    qst_ref[...] = q
    enc_ref[...] = enc
    idx_ref[...] = idx.astype(jnp.int32)

    ones_b = jnp.ones((1, _R), jnp.bfloat16)
    hist_ref[...] += jnp.dot(ones_b, enc_b, preferred_element_type=jnp.float32)
    sse_ref[0, 0] += jnp.sum(x * x) + jnp.sum(dmin)

    @pl.when(t == pl.num_programs(0) - 1)
    def _finish():
        mse = sse_ref[0, 0] / (n_rows * _E)
        loss_ref[...] = jnp.broadcast_to(mse + _COMMIT * mse, (1, 1))
        p = hist_ref[...] / n_rows
        ent = -jnp.sum(p * jnp.log(p + _EPS), keepdims=True)
        perp_ref[...] = jnp.exp(ent).reshape(1, 1)


def kernel(x, w, is_training):
    lead_shape = x.shape[:-1]
    xf = x.reshape(-1, _E)
    n = xf.shape[0]
    grid = n // _R

    qst, enc, idx, loss, perp = pl.pallas_call(
        _vq_tile,
        grid=(grid,),
        in_specs=[
            pl.BlockSpec((_R, _E), lambda t: (t, 0)),
            pl.BlockSpec((_E, _E), lambda t: (0, 0)),
            pl.BlockSpec((_E, _E), lambda t: (0, 0)),
        ],
        out_specs=[
            pl.BlockSpec((_R, _E), lambda t: (t, 0)),
            pl.BlockSpec((_R, _E), lambda t: (t, 0)),
            pl.BlockSpec((_R, 1), lambda t: (t, 0)),
            pl.BlockSpec((1, 1), lambda t: (0, 0)),
            pl.BlockSpec((1, 1), lambda t: (0, 0)),
        ],
        out_shape=[
            jax.ShapeDtypeStruct((n, _E), jnp.float32),
            jax.ShapeDtypeStruct((n, _E), jnp.float32),
            jax.ShapeDtypeStruct((n, 1), jnp.int32),
            jax.ShapeDtypeStruct((1, 1), jnp.float32),
            jax.ShapeDtypeStruct((1, 1), jnp.float32),
        ],
        scratch_shapes=[
            pltpu.VMEM((1, _E), jnp.float32),
            pltpu.SMEM((1, 1), jnp.float32),
            pltpu.VMEM((1, _E), jnp.float32),
            pltpu.VMEM((_E, _E), jnp.bfloat16),
            pltpu.VMEM((_R, _E), jnp.float32),
        ],
    )(xf, w, w.T)

    return (qst.reshape(x.shape), loss[0, 0], perp[0, 0], enc,
            idx.reshape(lead_shape))
